# baseline (speedup 1.0000x reference)
import jax
import jax.numpy as jnp
from jax.experimental import pallas as pl
from jax.experimental.pallas import tpu as pltpu


def _moe_kernel(idx_ref, w_ref, x_ref, W1_ref, b1_ref, W2_ref, b2_ref, out_ref):
    b = pl.program_id(0)
    hw = out_ref.shape[2]
    xb = x_ref[0]
    e0 = idx_ref[b, 0]
    e1 = idx_ref[b, 1]
    w0 = w_ref[b, 0]
    w1 = w_ref[b, 1]

    w1cat = jnp.concatenate([W1_ref[e0], W1_ref[e1]], axis=0)
    b1cat = jnp.concatenate([b1_ref[e0], b1_ref[e1]], axis=0)
    h = jnp.dot(w1cat, xb, preferred_element_type=jnp.float32) + b1cat[:, None]
    h = jnp.maximum(h, 0.0).astype(jnp.bfloat16)

    w2cat = jnp.concatenate(
        [w0 * W2_ref[e0], w1 * W2_ref[e1]], axis=1
    ).astype(jnp.bfloat16)
    b2mix = w0 * b2_ref[e0] + w1 * b2_ref[e1]
    y = jnp.dot(w2cat, h, preferred_element_type=jnp.float32) + b2mix[:, None]
    out_ref[0] = y


def kernel(x, weights, indices, W1, b1, W2, b2):
    B, C_IN, H, W_SP = x.shape
    E, HIDDEN, _ = W1.shape
    C_OUT = W2.shape[1]
    HW = H * W_SP
    x3 = x.reshape(B, C_IN, HW).astype(jnp.bfloat16)
    W1b = W1.astype(jnp.bfloat16)

    grid_spec = pltpu.PrefetchScalarGridSpec(
        num_scalar_prefetch=2,
        grid=(B,),
        in_specs=[
            pl.BlockSpec((1, C_IN, HW), lambda b, idx, w: (b, 0, 0)),
            pl.BlockSpec((E, HIDDEN, C_IN), lambda b, idx, w: (0, 0, 0)),
            pl.BlockSpec((E, HIDDEN), lambda b, idx, w: (0, 0)),
            pl.BlockSpec((E, C_OUT, HIDDEN), lambda b, idx, w: (0, 0, 0)),
            pl.BlockSpec((E, C_OUT), lambda b, idx, w: (0, 0)),
        ],
        out_specs=pl.BlockSpec((1, C_OUT, HW), lambda b, idx, w: (b, 0, 0)),
    )
    out = pl.pallas_call(
        _moe_kernel,
        grid_spec=grid_spec,
        out_shape=jax.ShapeDtypeStruct((B, C_OUT, HW), jnp.float32),
    )(indices, weights, x3, W1b, b1, W2, b2)
    return out.reshape(B, C_OUT, H, W_SP)


# speedup vs baseline: 1.3951x; 1.3951x over previous
import jax
import jax.numpy as jnp
from jax.experimental import pallas as pl
from jax.experimental.pallas import tpu as pltpu

_IMGS_PER_STEP = 8


def _moe_kernel(idx_ref, w_ref, x_ref, W1_ref, b1_ref, W2_ref, b2_ref, out_ref):
    g = pl.program_id(0)
    for i in range(x_ref.shape[0]):
        b = g * x_ref.shape[0] + i
        xb = x_ref[i].astype(jnp.bfloat16)
        e0 = idx_ref[b, 0]
        e1 = idx_ref[b, 1]
        w0 = w_ref[b, 0]
        w1 = w_ref[b, 1]

        w1cat = jnp.concatenate([W1_ref[e0], W1_ref[e1]], axis=0)
        b1cat = jnp.concatenate([b1_ref[e0], b1_ref[e1]], axis=0)
        h = jnp.dot(w1cat, xb, preferred_element_type=jnp.float32) + b1cat[:, None]
        h = jnp.maximum(h, 0.0).astype(jnp.bfloat16)

        w2cat = jnp.concatenate(
            [w0 * W2_ref[e0], w1 * W2_ref[e1]], axis=1
        ).astype(jnp.bfloat16)
        b2mix = w0 * b2_ref[e0] + w1 * b2_ref[e1]
        y = jnp.dot(w2cat, h, preferred_element_type=jnp.float32) + b2mix[:, None]
        out_ref[i] = y


def kernel(x, weights, indices, W1, b1, W2, b2):
    B, C_IN, H, W_SP = x.shape
    E, HIDDEN, _ = W1.shape
    C_OUT = W2.shape[1]
    HW = H * W_SP
    G = _IMGS_PER_STEP
    x3 = x.reshape(B, C_IN, HW)
    W1b = W1.astype(jnp.bfloat16)

    grid_spec = pltpu.PrefetchScalarGridSpec(
        num_scalar_prefetch=2,
        grid=(B // G,),
        in_specs=[
            pl.BlockSpec((G, C_IN, HW), lambda b, idx, w: (b, 0, 0)),
            pl.BlockSpec((E, HIDDEN, C_IN), lambda b, idx, w: (0, 0, 0)),
            pl.BlockSpec((E, HIDDEN), lambda b, idx, w: (0, 0)),
            pl.BlockSpec((E, C_OUT, HIDDEN), lambda b, idx, w: (0, 0, 0)),
            pl.BlockSpec((E, C_OUT), lambda b, idx, w: (0, 0)),
        ],
        out_specs=pl.BlockSpec((G, C_OUT, HW), lambda b, idx, w: (b, 0, 0)),
    )
    out = pl.pallas_call(
        _moe_kernel,
        grid_spec=grid_spec,
        out_shape=jax.ShapeDtypeStruct((B, C_OUT, HW), jnp.float32),
    )(indices, weights, x3, W1b, b1, W2, b2)
    return out.reshape(B, C_OUT, H, W_SP)
